# baseline (speedup 1.0000x reference)
import functools

import jax
import jax.numpy as jnp
from jax.experimental import pallas as pl
from jax.experimental.pallas import tpu as pltpu

E = 16
TOP_K = 2
H = 1024
FF = 2048
FFC = 2048
NF = FF // FFC


def _moe_body(x_ref, gate_ref, wg_ref, wu_ref, w2_ref, out_ref, comb_ref):
    e = pl.program_id(0)
    f = pl.program_id(1)
    T = x_ref.shape[0]

    @pl.when((e == 0) & (f == 0))
    def _router():
        x = x_ref[...]
        logits = jax.lax.dot_general(
            x, gate_ref[...], (((1,), (1,)), ((), ())),
            preferred_element_type=jnp.float32)
        ids = jax.lax.broadcasted_iota(jnp.int32, (T, E), 1)
        m1 = jnp.max(logits, axis=-1, keepdims=True)
        idx1 = jnp.argmax(logits, axis=-1)[:, None]
        oh1 = (ids == idx1)
        masked = jnp.where(oh1, -jnp.inf, logits)
        m2 = jnp.max(masked, axis=-1, keepdims=True)
        idx2 = jnp.argmax(masked, axis=-1)[:, None]
        oh2 = (ids == idx2)
        e2 = jnp.exp(m2 - m1)
        denom = 1.0 + e2
        comb = jnp.where(oh1, 1.0 / denom, 0.0) + jnp.where(oh2, e2 / denom, 0.0)
        comb_ref[...] = comb
        out_ref[...] = jnp.zeros_like(out_ref)

    x = x_ref[...]
    g = jax.lax.dot_general(x, wg_ref[0], (((1,), (1,)), ((), ())),
                            preferred_element_type=jnp.float32)
    u = jax.lax.dot_general(x, wu_ref[0], (((1,), (1,)), ((), ())),
                            preferred_element_type=jnp.float32)
    act = (g * jax.nn.sigmoid(g)) * u
    ye = jax.lax.dot_general(act, w2_ref[0], (((1,), (1,)), ((), ())),
                             preferred_element_type=jnp.float32)
    oh_e = (jax.lax.broadcasted_iota(jnp.int32, (E, 1), 0) == e).astype(jnp.float32)
    c = jax.lax.dot_general(comb_ref[...], oh_e, (((1,), (0,)), ((), ())),
                            preferred_element_type=jnp.float32)
    out_ref[...] += ye * c


@jax.jit
def kernel(hidden_states, gate_w, ws, w2s):
    b, s, h = hidden_states.shape
    x = hidden_states.reshape(-1, h)
    T = x.shape[0]

    grid = (E, NF)
    out = pl.pallas_call(
        _moe_body,
        grid=grid,
        in_specs=[
            pl.BlockSpec((T, H), lambda e, f: (0, 0)),
            pl.BlockSpec((E, H), lambda e, f: (0, 0)),
            pl.BlockSpec((1, FFC, H), lambda e, f: (e, f, 0)),
            pl.BlockSpec((1, FFC, H), lambda e, f: (e, NF + f, 0)),
            pl.BlockSpec((1, H, FFC), lambda e, f: (e, 0, f)),
        ],
        out_specs=pl.BlockSpec((T, H), lambda e, f: (0, 0)),
        out_shape=jax.ShapeDtypeStruct((T, H), jnp.float32),
        scratch_shapes=[pltpu.VMEM((T, E), jnp.float32)],
        compiler_params=pltpu.CompilerParams(
            dimension_semantics=("arbitrary", "arbitrary"),
        ),
    )(x, gate_w, ws, ws, w2s)
    return out.reshape(b, s, h)


# speedup vs baseline: 1.0234x; 1.0234x over previous
import functools

import jax
import jax.numpy as jnp
from jax.experimental import pallas as pl
from jax.experimental.pallas import tpu as pltpu

E = 16
TOP_K = 2
H = 1024
FF = 2048
FFC = 1024
NF = FF // FFC


def _moe_body(x_ref, gate_ref, wg_ref, wu_ref, w2_ref, out_ref, comb_ref):
    e = pl.program_id(0)
    f = pl.program_id(1)
    T = x_ref.shape[0]

    @pl.when((e == 0) & (f == 0))
    def _router():
        x = x_ref[...]
        logits = jax.lax.dot_general(
            x, gate_ref[...], (((1,), (1,)), ((), ())),
            preferred_element_type=jnp.float32)
        ids = jax.lax.broadcasted_iota(jnp.int32, (T, E), 1)
        m1 = jnp.max(logits, axis=-1, keepdims=True)
        idx1 = jnp.argmax(logits, axis=-1)[:, None]
        oh1 = (ids == idx1)
        masked = jnp.where(oh1, -jnp.inf, logits)
        m2 = jnp.max(masked, axis=-1, keepdims=True)
        idx2 = jnp.argmax(masked, axis=-1)[:, None]
        oh2 = (ids == idx2)
        e2 = jnp.exp(m2 - m1)
        denom = 1.0 + e2
        comb = jnp.where(oh1, 1.0 / denom, 0.0) + jnp.where(oh2, e2 / denom, 0.0)
        comb_ref[...] = comb
        out_ref[...] = jnp.zeros_like(out_ref)

    x = x_ref[...]
    g = jax.lax.dot_general(x, wg_ref[0], (((1,), (1,)), ((), ())),
                            preferred_element_type=jnp.float32)
    u = jax.lax.dot_general(x, wu_ref[0], (((1,), (1,)), ((), ())),
                            preferred_element_type=jnp.float32)
    act = (g * jax.nn.sigmoid(g)) * u
    ye = jax.lax.dot_general(act, w2_ref[0], (((1,), (1,)), ((), ())),
                             preferred_element_type=jnp.float32)
    oh_e = (jax.lax.broadcasted_iota(jnp.int32, (E, 1), 0) == e).astype(jnp.float32)
    c = jax.lax.dot_general(comb_ref[...], oh_e, (((1,), (0,)), ((), ())),
                            preferred_element_type=jnp.float32)
    out_ref[...] += ye * c


@jax.jit
def kernel(hidden_states, gate_w, ws, w2s):
    b, s, h = hidden_states.shape
    x = hidden_states.reshape(-1, h)
    T = x.shape[0]

    grid = (E, NF)
    out = pl.pallas_call(
        _moe_body,
        grid=grid,
        in_specs=[
            pl.BlockSpec((T, H), lambda e, f: (0, 0)),
            pl.BlockSpec((E, H), lambda e, f: (0, 0)),
            pl.BlockSpec((1, FFC, H), lambda e, f: (e, f, 0)),
            pl.BlockSpec((1, FFC, H), lambda e, f: (e, NF + f, 0)),
            pl.BlockSpec((1, H, FFC), lambda e, f: (e, 0, f)),
        ],
        out_specs=pl.BlockSpec((T, H), lambda e, f: (0, 0)),
        out_shape=jax.ShapeDtypeStruct((T, H), jnp.float32),
        scratch_shapes=[pltpu.VMEM((T, E), jnp.float32)],
        compiler_params=pltpu.CompilerParams(
            dimension_semantics=("arbitrary", "arbitrary"),
        ),
    )(x, gate_w, ws, ws, w2s)
    return out.reshape(b, s, h)
